# Initial kernel scaffold; baseline (speedup 1.0000x reference)
import jax
import jax.numpy as jnp
from jax.experimental import pallas as pl


def kernel(inputs, embedding):
    raise NotImplementedError("write your pallas kernel here")


# speedup vs baseline: 1.4077x; 1.4077x over previous
import functools

import jax
import jax.numpy as jnp
from jax import lax
from jax.experimental import pallas as pl
from jax.experimental.pallas import tpu as pltpu
from jax.experimental.pallas import tpu_sc as plsc

COMMITMENT_COST = 0.25

TM = 1024
TN = 2048


def _bf16_round(v):
    return v.astype(jnp.bfloat16).astype(jnp.float32)


def _argmin_body(nt, mt, et_ref, xt_ref, x2_ref, n2_ref, idx_ref, loss_ref,
                 best_val, best_f32, best_idx, acc):
    m = pl.program_id(0)
    n = pl.program_id(1)
    et = et_ref[...]
    xt = xt_ref[0]
    s = jnp.dot(et, xt, preferred_element_type=jnp.float32)
    x2 = x2_ref[0, 0, :]
    n2 = n2_ref[0, 0, :]
    dist = (x2[None, :] - 2.0 * s) + n2[:, None]
    local_min = jnp.min(dist, axis=0)
    local_arg = jnp.argmin(dist, axis=0).astype(jnp.int32) + n * TN

    @pl.when(n == 0)
    def _():
        best_val[...] = _bf16_round(local_min)
        best_f32[...] = local_min
        best_idx[...] = local_arg

    @pl.when(n > 0)
    def _():
        bv = best_val[...]
        bi = best_idx[...]
        take = local_min < bv
        best_val[...] = _bf16_round(jnp.where(take, local_min, bv))
        best_f32[...] = jnp.where(take, local_min, best_f32[...])
        best_idx[...] = jnp.where(take, local_arg, bi)

    @pl.when((m == 0) & (n == 0))
    def _():
        acc[0] = 0.0

    @pl.when(n == nt - 1)
    def _():
        idx_ref[0, 0, :] = best_idx[...]
        acc[0] = acc[0] + jnp.sum(best_f32[...])

    @pl.when((m == mt - 1) & (n == nt - 1))
    def _():
        loss_ref[0] = acc[0]


def _argmin_call(et, xt, x2, n2):
    num_codes, d = et.shape
    batches = xt.shape[0]
    tokens = batches * xt.shape[2]
    mt = tokens // TM
    nt = num_codes // TN
    return pl.pallas_call(
        functools.partial(_argmin_body, nt, mt),
        grid=(mt, nt),
        in_specs=[
            pl.BlockSpec((TN, d), lambda m, n: (n, 0)),
            pl.BlockSpec((1, d, TM), lambda m, n: (m, 0, 0)),
            pl.BlockSpec((1, 1, TM), lambda m, n: (m, 0, 0)),
            pl.BlockSpec((1, 1, TN), lambda m, n: (n, 0, 0)),
        ],
        out_specs=[
            pl.BlockSpec((1, 1, TM), lambda m, n: (m, 0, 0)),
            pl.BlockSpec(memory_space=pltpu.SMEM),
        ],
        out_shape=[
            jax.ShapeDtypeStruct((mt, 1, TM), jnp.int32),
            jax.ShapeDtypeStruct((1,), jnp.float32),
        ],
        scratch_shapes=[
            pltpu.VMEM((TM,), jnp.float32),
            pltpu.VMEM((TM,), jnp.float32),
            pltpu.VMEM((TM,), jnp.int32),
            pltpu.SMEM((1,), jnp.float32),
        ],
    )(et, xt, x2, n2)


def _sc_gather(table, idx):
    info = plsc.get_sparse_core_info()
    nc, ns = info.num_cores, info.num_subcores
    nw = nc * ns
    b = idx.shape[0]
    d = table.shape[1]
    bpw = b // nw
    mesh = plsc.VectorSubcoreMesh(core_axis_name="c", subcore_axis_name="s")

    @functools.partial(
        pl.kernel,
        mesh=mesh,
        out_type=jax.ShapeDtypeStruct((b, d), jnp.float32),
        scratch_types=[
            pltpu.VMEM((bpw,), jnp.int32),
            pltpu.VMEM((bpw, d), jnp.float32),
            pltpu.SemaphoreType.DMA,
        ],
    )
    def k(table_hbm, idx_hbm, out_hbm, idx_v, rows_v, sem):
        wid = lax.axis_index("s") * nc + lax.axis_index("c")
        base = wid * bpw
        pltpu.sync_copy(idx_hbm.at[pl.ds(base, bpw)], idx_v)
        pltpu.async_copy(table_hbm.at[idx_v], rows_v, sem).wait()
        pltpu.sync_copy(rows_v, out_hbm.at[pl.ds(base, bpw)])

    return k(table, idx)


def kernel(inputs, embedding):
    b, c, h, w = inputs.shape
    xt = inputs.reshape(b, c, h * w)
    et = embedding.T
    x2 = jnp.sum(jnp.transpose(inputs, (0, 2, 3, 1)) ** 2, axis=3)
    x2 = x2.reshape(b, 1, h * w)
    n2 = jnp.sum(embedding ** 2, axis=0).reshape(-1, 1, TN)
    idx3, loss_sum = _argmin_call(et, xt, x2, n2)
    idx_flat = idx3.reshape(-1)
    quant_flat = _sc_gather(et, idx_flat)
    quantized = jnp.transpose(quant_flat.reshape(b, h, w, c), (0, 3, 1, 2))
    commitment_loss = loss_sum[0] * (COMMITMENT_COST / (b * c * h * w))
    indices = idx_flat.reshape(b, h, w)
    return (quantized, commitment_loss, indices)
